# baseline (speedup 1.0000x reference)
import jax
import jax.numpy as jnp
from jax import lax
from jax.experimental import pallas as pl
from jax.experimental.pallas import tpu as pltpu
from jax.experimental.pallas import tpu_sc as plsc

N = 10000
E = 160000
IN_CH = 256
HID = 512
OUT_CH = 16
HALF = IN_CH // 2
LW = 128

NC, NS = 2, 16
CHUNK = 128
PAD_ROWS = 240
NP_ROWS = 10240
E_PAD = 163840

INIT_ROWS_PER_TILE = NP_ROWS // NS
ROWS_A, ROWS_B = 632, N - 15 * 632

_MESH = plsc.VectorSubcoreMesh(
    core_axis_name="c", subcore_axis_name="s", num_cores=NC, num_subcores=NS
)


def _al8(v):
    return pl.multiple_of(v, 8)


def _tile_rows_copy(s, get_src, get_dst):
    @pl.when(s < NS - 1)
    def _():
        pltpu.sync_copy(get_src(pl.ds(_al8(s * ROWS_A), ROWS_A)),
                        get_dst(pl.ds(_al8(s * ROWS_A), ROWS_A)))

    @pl.when(s == NS - 1)
    def _():
        pltpu.sync_copy(get_src(pl.ds((NS - 1) * ROWS_A, ROWS_B)),
                        get_dst(pl.ds((NS - 1) * ROWS_A, ROWS_B)))


def _init_zero(s, zeros_hbm, acc_sh):
    r0 = _al8(s * INIT_ROWS_PER_TILE)
    pltpu.sync_copy(zeros_hbm.at[pl.ds(r0, INIT_ROWS_PER_TILE)],
                    acc_sh.at[pl.ds(r0, INIT_ROWS_PER_TILE)])


_NW_CHUNKS = E_PAD // (NC * NS) // CHUNK
_NS_CHUNKS = E_PAD // NS // CHUNK


def _deg_body(dst_hbm, ones_hbm, zeros_hbm, deg0_hbm, deg1_hbm,
              idxs, ones_v, acc_sh, sem0, sem1):
    c = lax.axis_index("c")
    s = lax.axis_index("s")
    _init_zero(s, zeros_hbm, acc_sh)
    pltpu.sync_copy(ones_hbm, ones_v)
    pltpu.sync_copy(dst_hbm.at[c * NS + s], idxs)
    plsc.subcore_barrier()

    n = _NW_CHUNKS
    sems = (sem0, sem1)

    def wait(b):
        pltpu.make_async_copy(ones_v, acc_sh.at[idxs.at[0]], sems[b]).wait()

    def body(t, carry):
        for p in (0, 1):
            @pl.when(t > 0)
            def _():
                wait(p)

            pltpu.async_copy(ones_v, acc_sh.at[idxs.at[2 * t + p]],
                             sems[p], add=True)
        return carry

    lax.fori_loop(0, n // 2, body, 0)
    wait(0)
    wait(1)
    plsc.subcore_barrier()

    @pl.when(c == 0)
    def _():
        _tile_rows_copy(s, lambda d: acc_sh.at[d], lambda d: deg0_hbm.at[d])

    @pl.when(c == 1)
    def _():
        _tile_rows_copy(s, lambda d: acc_sh.at[d], lambda d: deg1_hbm.at[d])


_deg_kernel = pl.kernel(
    _deg_body,
    out_type=(
        jax.ShapeDtypeStruct((N, LW), jnp.float32),
        jax.ShapeDtypeStruct((N, LW), jnp.float32),
    ),
    mesh=_MESH,
    scratch_types=[
        pltpu.VMEM((_NW_CHUNKS, CHUNK), jnp.int32),
        pltpu.VMEM((CHUNK, LW), jnp.float32),
        pltpu.VMEM_SHARED((NP_ROWS, LW), jnp.float32),
        pltpu.SemaphoreType.DMA,
        pltpu.SemaphoreType.DMA,
    ],
)


_CH = 64
_NR = 5


def _edge_sweep(n, base, srcf_hbm, dstf_hbm, gbufs, sbufs, rows,
                gi, si, sg, ss, acc_sh, gsrc_start, gsrc_wait):
    R = _NR

    def gidx_load(j, b):
        off = _al8(base + j * _CH)
        pltpu.async_copy(srcf_hbm.at[pl.ds(off, _CH)], gbufs[b], gi[b])

    def gidx_wait(b):
        pltpu.make_async_copy(srcf_hbm.at[pl.ds(0, _CH)], gbufs[b],
                              gi[b]).wait()

    def sidx_load(j, b):
        off = _al8(base + j * _CH)
        pltpu.async_copy(dstf_hbm.at[pl.ds(off, _CH)], sbufs[b], si[b])

    def sidx_wait(b):
        pltpu.make_async_copy(dstf_hbm.at[pl.ds(0, _CH)], sbufs[b],
                              si[b]).wait()

    def gather_start(b):
        gsrc_start(gbufs[b], rows[b], sg[b])

    def gather_wait(b):
        gsrc_wait(gbufs[b], rows[b], sg[b])

    def scat_start(b):
        pltpu.async_copy(rows[b], acc_sh.at[sbufs[b]], ss[b], add=True)

    def scat_wait(b):
        pltpu.make_async_copy(rows[b], acc_sh.at[sbufs[b]], ss[b]).wait()

    for k in range(R):
        gidx_load(k, k)
    for k in range(R - 1):
        sidx_load(k, k)
    for k in range(R - 1):
        gidx_wait(k)
        gather_start(k)

    def step(j, b):
        gather_wait(b)
        sidx_wait(b)
        scat_start(b)
        nb = (b + R - 1) % R

        @pl.when(j >= 1)
        def _():
            scat_wait(nb)

        @pl.when(j + R - 1 < n)
        def _():
            gidx_wait(nb)
            gather_start(nb)
            sidx_load(j + R - 1, nb)

        @pl.when(j + R < n)
        def _():
            gidx_load(j + R, b)

    def body(t, carry):
        for b in range(R):
            step(R * t + b, b)
        return carry

    lax.fori_loop(0, n // R, body, 0)
    scat_wait((n - 1) % R)


def _agg1_body(y_lo_hbm, y_hi_hbm, srcf_hbm, dstf_hbm, agg_lo_hbm, agg_hi_hbm,
               gbufs, sbufs, rows, acc_sh, gi, si, sg, ss):
    c = lax.axis_index("c")
    s = lax.axis_index("s")

    @pl.when(c == 0)
    def _():
        _tile_rows_copy(s, lambda d: y_lo_hbm.at[d], lambda d: acc_sh.at[d])

    @pl.when(c == 1)
    def _():
        _tile_rows_copy(s, lambda d: y_hi_hbm.at[d], lambda d: acc_sh.at[d])

    plsc.subcore_barrier()

    def gsrc_start(idx_ref, row_ref, sem):
        @pl.when(c == 0)
        def _():
            pltpu.async_copy(y_lo_hbm.at[idx_ref], row_ref, sem)

        @pl.when(c == 1)
        def _():
            pltpu.async_copy(y_hi_hbm.at[idx_ref], row_ref, sem)

    def gsrc_wait(idx_ref, row_ref, sem):
        pltpu.make_async_copy(y_lo_hbm.at[idx_ref], row_ref, sem).wait()

    base = s * (E_PAD // NS)
    _edge_sweep(E_PAD // NS // _CH, base, srcf_hbm, dstf_hbm,
                gbufs, sbufs, rows, gi, si, sg, ss,
                acc_sh, gsrc_start, gsrc_wait)
    plsc.subcore_barrier()

    @pl.when(c == 0)
    def _():
        _tile_rows_copy(s, lambda d: acc_sh.at[d], lambda d: agg_lo_hbm.at[d])

    @pl.when(c == 1)
    def _():
        _tile_rows_copy(s, lambda d: acc_sh.at[d], lambda d: agg_hi_hbm.at[d])


_agg1_kernel = pl.kernel(
    _agg1_body,
    out_type=(
        jax.ShapeDtypeStruct((N, HALF), jnp.float32),
        jax.ShapeDtypeStruct((N, HALF), jnp.float32),
    ),
    mesh=_MESH,
    scratch_types=[
        [pltpu.VMEM((_CH,), jnp.int32)] * _NR,
        [pltpu.VMEM((_CH,), jnp.int32)] * _NR,
        [pltpu.VMEM((_CH, HALF), jnp.float32)] * _NR,
        pltpu.VMEM_SHARED((NP_ROWS, HALF), jnp.float32),
        [pltpu.SemaphoreType.DMA] * _NR,
        [pltpu.SemaphoreType.DMA] * _NR,
        [pltpu.SemaphoreType.DMA] * _NR,
        [pltpu.SemaphoreType.DMA] * _NR,
    ],
)


def _agg2_body(z_hbm, zeros_hbm, srcf_hbm, dstf_hbm, s0_hbm, s1_hbm,
               gbufs, sbufs, rows, acc_sh, gi, si, sg, ss):
    c = lax.axis_index("c")
    s = lax.axis_index("s")
    _init_zero(s, zeros_hbm, acc_sh)
    plsc.subcore_barrier()

    def gsrc_start(idx_ref, row_ref, sem):
        pltpu.async_copy(z_hbm.at[idx_ref], row_ref, sem)

    def gsrc_wait(idx_ref, row_ref, sem):
        pltpu.make_async_copy(z_hbm.at[idx_ref], row_ref, sem).wait()

    base = (c * NS + s) * (E_PAD // (NC * NS))
    _edge_sweep(E_PAD // (NC * NS) // _CH, base, srcf_hbm, dstf_hbm,
                gbufs, sbufs, rows, gi, si, sg, ss,
                acc_sh, gsrc_start, gsrc_wait)
    plsc.subcore_barrier()

    @pl.when(c == 0)
    def _():
        _tile_rows_copy(s, lambda d: acc_sh.at[d], lambda d: s0_hbm.at[d])

    @pl.when(c == 1)
    def _():
        _tile_rows_copy(s, lambda d: acc_sh.at[d], lambda d: s1_hbm.at[d])


_agg2_kernel = pl.kernel(
    _agg2_body,
    out_type=(
        jax.ShapeDtypeStruct((N, LW), jnp.float32),
        jax.ShapeDtypeStruct((N, LW), jnp.float32),
    ),
    mesh=_MESH,
    scratch_types=[
        [pltpu.VMEM((_CH,), jnp.int32)] * _NR,
        [pltpu.VMEM((_CH,), jnp.int32)] * _NR,
        [pltpu.VMEM((_CH, LW), jnp.float32)] * _NR,
        pltpu.VMEM_SHARED((NP_ROWS, LW), jnp.float32),
        [pltpu.SemaphoreType.DMA] * _NR,
        [pltpu.SemaphoreType.DMA] * _NR,
        [pltpu.SemaphoreType.DMA] * _NR,
        [pltpu.SemaphoreType.DMA] * _NR,
    ],
)


_BM = 1000


def _prep_body(deg0, deg1, x, y_lo, y_hi, dinv2):
    deg = deg0[:, :1] + deg1[:, :1] + 1.0
    dinv = lax.rsqrt(deg)
    y = x[...] * dinv
    y_lo[...] = y[:, :HALF]
    y_hi[...] = y[:, HALF:]
    dinv2[...] = dinv


def _prep_call(deg0, deg1, x):
    return pl.pallas_call(
        _prep_body,
        grid=(N // _BM,),
        in_specs=[
            pl.BlockSpec((_BM, LW), lambda i: (i, 0)),
            pl.BlockSpec((_BM, LW), lambda i: (i, 0)),
            pl.BlockSpec((_BM, IN_CH), lambda i: (i, 0)),
        ],
        out_specs=[
            pl.BlockSpec((_BM, HALF), lambda i: (i, 0)),
            pl.BlockSpec((_BM, HALF), lambda i: (i, 0)),
            pl.BlockSpec((_BM, 1), lambda i: (i, 0)),
        ],
        out_shape=[
            jax.ShapeDtypeStruct((N, HALF), jnp.float32),
            jax.ShapeDtypeStruct((N, HALF), jnp.float32),
            jax.ShapeDtypeStruct((N, 1), jnp.float32),
        ],
    )(deg0, deg1, x)


def _bf(v):
    return v.astype(jnp.bfloat16)


def _main_body(agg_lo, agg_hi, dinv2, w1l, w1h, b1r, w2p, z_out):
    d = dinv2[...]
    h = (jnp.dot(_bf(agg_lo[...] * d), _bf(w1l[...]),
                 preferred_element_type=jnp.float32)
         + jnp.dot(_bf(agg_hi[...] * d), _bf(w1h[...]),
                   preferred_element_type=jnp.float32)
         + b1r[...])
    h = jnp.maximum(h, 0.0)
    z_out[...] = jnp.dot(_bf(h), _bf(w2p[...]),
                         preferred_element_type=jnp.float32) * d


def _main_call(agg_lo, agg_hi, dinv2, w1l, w1h, b1r, w2p):
    return pl.pallas_call(
        _main_body,
        grid=(N // _BM,),
        in_specs=[
            pl.BlockSpec((_BM, HALF), lambda i: (i, 0)),
            pl.BlockSpec((_BM, HALF), lambda i: (i, 0)),
            pl.BlockSpec((_BM, 1), lambda i: (i, 0)),
            pl.BlockSpec((HALF, HID), lambda i: (0, 0)),
            pl.BlockSpec((HALF, HID), lambda i: (0, 0)),
            pl.BlockSpec((1, HID), lambda i: (0, 0)),
            pl.BlockSpec((HID, LW), lambda i: (0, 0)),
        ],
        out_specs=pl.BlockSpec((_BM, LW), lambda i: (i, 0)),
        out_shape=jax.ShapeDtypeStruct((N, LW), jnp.float32),
    )(agg_lo, agg_hi, dinv2, w1l, w1h, b1r, w2p)


def _final_body(z, s0, s1, dinv2, b2r, out):
    res = (z[...] + s0[...] + s1[...]) * dinv2[...]
    out[...] = res[:, :OUT_CH] + b2r[...]


def _final_call(z, s0, s1, dinv2, b2r):
    return pl.pallas_call(
        _final_body,
        grid=(N // _BM,),
        in_specs=[
            pl.BlockSpec((_BM, LW), lambda i: (i, 0)),
            pl.BlockSpec((_BM, LW), lambda i: (i, 0)),
            pl.BlockSpec((_BM, LW), lambda i: (i, 0)),
            pl.BlockSpec((_BM, 1), lambda i: (i, 0)),
            pl.BlockSpec((1, OUT_CH), lambda i: (0, 0)),
        ],
        out_specs=pl.BlockSpec((_BM, OUT_CH), lambda i: (i, 0)),
        out_shape=jax.ShapeDtypeStruct((N, OUT_CH), jnp.float32),
    )(z, s0, s1, dinv2, b2r)


def kernel(x, edge_index, W1, b1, W2, b2):
    ei = edge_index.astype(jnp.int32)
    pad_n = E_PAD - E
    src = jnp.concatenate([ei[0], jnp.zeros((pad_n,), jnp.int32)])
    dst = jnp.concatenate(
        [ei[1], N + (jnp.arange(pad_n, dtype=jnp.int32) % PAD_ROWS)])

    onehot = jnp.zeros((CHUNK, LW), jnp.float32).at[:, 0].set(1.0)
    zeros_np = jnp.zeros((NP_ROWS, LW), jnp.float32)
    w2p = jnp.zeros((HID, LW), jnp.float32).at[:, :OUT_CH].set(W2)

    dst_w = dst.reshape(NC * NS, _NW_CHUNKS, CHUNK)

    deg0, deg1 = _deg_kernel(dst_w, onehot, zeros_np)
    y_lo, y_hi, dinv2 = _prep_call(deg0, deg1, x)
    agg_lo, agg_hi = _agg1_kernel(y_lo, y_hi, src, dst)
    z = _main_call(agg_lo, agg_hi, dinv2,
                   W1[:HALF], W1[HALF:], b1.reshape(1, HID), w2p)
    s0, s1 = _agg2_kernel(z, zeros_np, src, dst)
    return _final_call(z, s0, s1, dinv2, b2.reshape(1, OUT_CH))


# speedup vs baseline: 2.4647x; 2.4647x over previous
import jax
import jax.numpy as jnp
from jax import lax
from jax.experimental import pallas as pl
from jax.experimental.pallas import tpu as pltpu
from jax.experimental.pallas import tpu_sc as plsc

N = 10000
E = 160000
IN_CH = 256
HID = 512
OUT_CH = 16
HALF = IN_CH // 2
LW = 128

NC, NS = 2, 16
CHUNK = 128
PAD_ROWS = 240
NP_ROWS = 10240
E_PAD = 163840

INIT_ROWS_PER_TILE = NP_ROWS // NS
ROWS_A, ROWS_B = 632, N - 15 * 632

_MESH = plsc.VectorSubcoreMesh(
    core_axis_name="c", subcore_axis_name="s", num_cores=NC, num_subcores=NS
)


def _al8(v):
    return pl.multiple_of(v, 8)


def _tile_rows_copy(s, get_src, get_dst):
    @pl.when(s < NS - 1)
    def _():
        pltpu.sync_copy(get_src(pl.ds(_al8(s * ROWS_A), ROWS_A)),
                        get_dst(pl.ds(_al8(s * ROWS_A), ROWS_A)))

    @pl.when(s == NS - 1)
    def _():
        pltpu.sync_copy(get_src(pl.ds((NS - 1) * ROWS_A, ROWS_B)),
                        get_dst(pl.ds((NS - 1) * ROWS_A, ROWS_B)))


def _init_zero(s, zeros_hbm, acc_sh):
    r0 = _al8(s * INIT_ROWS_PER_TILE)
    pltpu.sync_copy(zeros_hbm.at[pl.ds(r0, INIT_ROWS_PER_TILE)],
                    acc_sh.at[pl.ds(r0, INIT_ROWS_PER_TILE)])


_NW_CHUNKS = E_PAD // (NC * NS) // CHUNK
_NS_CHUNKS = E_PAD // NS // CHUNK


def _deg_body(dst_hbm, ones_hbm, zeros_hbm, deg0_hbm, deg1_hbm,
              idxs, ones_v, acc_sh, sem0, sem1):
    c = lax.axis_index("c")
    s = lax.axis_index("s")
    _init_zero(s, zeros_hbm, acc_sh)
    pltpu.sync_copy(ones_hbm, ones_v)
    pltpu.sync_copy(dst_hbm.at[c * NS + s], idxs)
    plsc.subcore_barrier()

    n = _NW_CHUNKS
    sems = (sem0, sem1)

    def wait(b):
        pltpu.make_async_copy(ones_v, acc_sh.at[idxs.at[0]], sems[b]).wait()

    def body(t, carry):
        for p in (0, 1):
            @pl.when(t > 0)
            def _():
                wait(p)

            pltpu.async_copy(ones_v, acc_sh.at[idxs.at[2 * t + p]],
                             sems[p], add=True)
        return carry

    lax.fori_loop(0, n // 2, body, 0)
    wait(0)
    wait(1)
    plsc.subcore_barrier()

    @pl.when(c == 0)
    def _():
        _tile_rows_copy(s, lambda d: acc_sh.at[d], lambda d: deg0_hbm.at[d])

    @pl.when(c == 1)
    def _():
        _tile_rows_copy(s, lambda d: acc_sh.at[d], lambda d: deg1_hbm.at[d])


_deg_kernel = pl.kernel(
    _deg_body,
    out_type=(
        jax.ShapeDtypeStruct((N, LW), jnp.float32),
        jax.ShapeDtypeStruct((N, LW), jnp.float32),
    ),
    mesh=_MESH,
    scratch_types=[
        pltpu.VMEM((_NW_CHUNKS, CHUNK), jnp.int32),
        pltpu.VMEM((CHUNK, LW), jnp.float32),
        pltpu.VMEM_SHARED((NP_ROWS, LW), jnp.float32),
        pltpu.SemaphoreType.DMA,
        pltpu.SemaphoreType.DMA,
    ],
)


_CH = 64
_NR = 5


def _edge_sweep(n, base, srcf_hbm, dstf_hbm, gbufs, sbufs, rows,
                gi, si, sg, ss, acc_sh, gsrc_start, gsrc_wait):
    R = _NR

    def gidx_load(j, b):
        off = _al8(base + j * _CH)
        pltpu.async_copy(srcf_hbm.at[pl.ds(off, _CH)], gbufs[b], gi[b])

    def gidx_wait(b):
        pltpu.make_async_copy(srcf_hbm.at[pl.ds(0, _CH)], gbufs[b],
                              gi[b]).wait()

    def sidx_load(j, b):
        off = _al8(base + j * _CH)
        pltpu.async_copy(dstf_hbm.at[pl.ds(off, _CH)], sbufs[b], si[b])

    def sidx_wait(b):
        pltpu.make_async_copy(dstf_hbm.at[pl.ds(0, _CH)], sbufs[b],
                              si[b]).wait()

    def gather_start(b):
        gsrc_start(gbufs[b], rows[b], sg[b])

    def gather_wait(b):
        gsrc_wait(gbufs[b], rows[b], sg[b])

    def scat_start(b):
        pltpu.async_copy(rows[b], acc_sh.at[sbufs[b]], ss[b], add=True)

    def scat_wait(b):
        pltpu.make_async_copy(rows[b], acc_sh.at[sbufs[b]], ss[b]).wait()

    for k in range(R):
        gidx_load(k, k)
    for k in range(R - 1):
        sidx_load(k, k)
    for k in range(R - 1):
        gidx_wait(k)
        gather_start(k)

    def step(j, b):
        gather_wait(b)
        sidx_wait(b)
        scat_start(b)
        nb = (b + R - 1) % R

        @pl.when(j >= 1)
        def _():
            scat_wait(nb)

        @pl.when(j + R - 1 < n)
        def _():
            gidx_wait(nb)
            gather_start(nb)
            sidx_load(j + R - 1, nb)

        @pl.when(j + R < n)
        def _():
            gidx_load(j + R, b)

    def body(t, carry):
        for b in range(R):
            step(R * t + b, b)
        return carry

    lax.fori_loop(0, n // R, body, 0)
    scat_wait((n - 1) % R)


def _agg1_body(y_lo_hbm, y_hi_hbm, srcf_hbm, dstf_hbm, agg_lo_hbm, agg_hi_hbm,
               gbufs, sbufs, rows, acc_sh, gi, si, sg, ss):
    c = lax.axis_index("c")
    s = lax.axis_index("s")

    @pl.when(c == 0)
    def _():
        _tile_rows_copy(s, lambda d: y_lo_hbm.at[d], lambda d: acc_sh.at[d])

    @pl.when(c == 1)
    def _():
        _tile_rows_copy(s, lambda d: y_hi_hbm.at[d], lambda d: acc_sh.at[d])

    plsc.subcore_barrier()

    def gsrc_start(idx_ref, row_ref, sem):
        @pl.when(c == 0)
        def _():
            pltpu.async_copy(y_lo_hbm.at[idx_ref], row_ref, sem)

        @pl.when(c == 1)
        def _():
            pltpu.async_copy(y_hi_hbm.at[idx_ref], row_ref, sem)

    def gsrc_wait(idx_ref, row_ref, sem):
        pltpu.make_async_copy(y_lo_hbm.at[idx_ref], row_ref, sem).wait()

    base = s * (E_PAD // NS)
    _edge_sweep(E_PAD // NS // _CH, base, srcf_hbm, dstf_hbm,
                gbufs, sbufs, rows, gi, si, sg, ss,
                acc_sh, gsrc_start, gsrc_wait)
    plsc.subcore_barrier()

    @pl.when(c == 0)
    def _():
        _tile_rows_copy(s, lambda d: acc_sh.at[d], lambda d: agg_lo_hbm.at[d])

    @pl.when(c == 1)
    def _():
        _tile_rows_copy(s, lambda d: acc_sh.at[d], lambda d: agg_hi_hbm.at[d])


_agg1_kernel = pl.kernel(
    _agg1_body,
    out_type=(
        jax.ShapeDtypeStruct((N, HALF), jnp.float32),
        jax.ShapeDtypeStruct((N, HALF), jnp.float32),
    ),
    mesh=_MESH,
    scratch_types=[
        [pltpu.VMEM((_CH,), jnp.int32)] * _NR,
        [pltpu.VMEM((_CH,), jnp.int32)] * _NR,
        [pltpu.VMEM((_CH, HALF), jnp.float32)] * _NR,
        pltpu.VMEM_SHARED((NP_ROWS, HALF), jnp.float32),
        [pltpu.SemaphoreType.DMA] * _NR,
        [pltpu.SemaphoreType.DMA] * _NR,
        [pltpu.SemaphoreType.DMA] * _NR,
        [pltpu.SemaphoreType.DMA] * _NR,
    ],
)


def _agg2_body(z_hbm, zeros_hbm, srcf_hbm, dstf_hbm, s0_hbm, s1_hbm,
               gbufs, sbufs, rows, acc_sh, gi, si, sg, ss):
    c = lax.axis_index("c")
    s = lax.axis_index("s")
    _init_zero(s, zeros_hbm, acc_sh)
    plsc.subcore_barrier()

    def gsrc_start(idx_ref, row_ref, sem):
        pltpu.async_copy(z_hbm.at[idx_ref], row_ref, sem)

    def gsrc_wait(idx_ref, row_ref, sem):
        pltpu.make_async_copy(z_hbm.at[idx_ref], row_ref, sem).wait()

    base = (c * NS + s) * (E_PAD // (NC * NS))
    _edge_sweep(E_PAD // (NC * NS) // _CH, base, srcf_hbm, dstf_hbm,
                gbufs, sbufs, rows, gi, si, sg, ss,
                acc_sh, gsrc_start, gsrc_wait)
    plsc.subcore_barrier()

    @pl.when(c == 0)
    def _():
        _tile_rows_copy(s, lambda d: acc_sh.at[d], lambda d: s0_hbm.at[d])

    @pl.when(c == 1)
    def _():
        _tile_rows_copy(s, lambda d: acc_sh.at[d], lambda d: s1_hbm.at[d])


_agg2_kernel = pl.kernel(
    _agg2_body,
    out_type=(
        jax.ShapeDtypeStruct((N, LW), jnp.float32),
        jax.ShapeDtypeStruct((N, LW), jnp.float32),
    ),
    mesh=_MESH,
    scratch_types=[
        [pltpu.VMEM((_CH,), jnp.int32)] * _NR,
        [pltpu.VMEM((_CH,), jnp.int32)] * _NR,
        [pltpu.VMEM((_CH, LW), jnp.float32)] * _NR,
        pltpu.VMEM_SHARED((NP_ROWS, LW), jnp.float32),
        [pltpu.SemaphoreType.DMA] * _NR,
        [pltpu.SemaphoreType.DMA] * _NR,
        [pltpu.SemaphoreType.DMA] * _NR,
        [pltpu.SemaphoreType.DMA] * _NR,
    ],
)


_BM = 1000


def _prep_body(deg0, deg1, x, y_lo, y_hi, dinv2):
    deg = deg0[:, :1] + deg1[:, :1] + 1.0
    dinv = lax.rsqrt(deg)
    y = x[...] * dinv
    y_lo[...] = y[:, :HALF]
    y_hi[...] = y[:, HALF:]
    dinv2[...] = dinv


def _prep_call(deg0, deg1, x):
    return pl.pallas_call(
        _prep_body,
        grid=(N // _BM,),
        in_specs=[
            pl.BlockSpec((_BM, LW), lambda i: (i, 0)),
            pl.BlockSpec((_BM, LW), lambda i: (i, 0)),
            pl.BlockSpec((_BM, IN_CH), lambda i: (i, 0)),
        ],
        out_specs=[
            pl.BlockSpec((_BM, HALF), lambda i: (i, 0)),
            pl.BlockSpec((_BM, HALF), lambda i: (i, 0)),
            pl.BlockSpec((_BM, 1), lambda i: (i, 0)),
        ],
        out_shape=[
            jax.ShapeDtypeStruct((N, HALF), jnp.float32),
            jax.ShapeDtypeStruct((N, HALF), jnp.float32),
            jax.ShapeDtypeStruct((N, 1), jnp.float32),
        ],
    )(deg0, deg1, x)


def _bf(v):
    return v.astype(jnp.bfloat16)


def _main_body(agg_lo, agg_hi, dinv2, w1l, w1h, b1r, w2p, z_out):
    d = dinv2[...]
    h = (jnp.dot(_bf(agg_lo[...] * d), _bf(w1l[...]),
                 preferred_element_type=jnp.float32)
         + jnp.dot(_bf(agg_hi[...] * d), _bf(w1h[...]),
                   preferred_element_type=jnp.float32)
         + b1r[...])
    h = jnp.maximum(h, 0.0)
    z_out[...] = jnp.dot(_bf(h), _bf(w2p[...]),
                         preferred_element_type=jnp.float32) * d


def _main_call(agg_lo, agg_hi, dinv2, w1l, w1h, b1r, w2p):
    return pl.pallas_call(
        _main_body,
        grid=(N // _BM,),
        in_specs=[
            pl.BlockSpec((_BM, HALF), lambda i: (i, 0)),
            pl.BlockSpec((_BM, HALF), lambda i: (i, 0)),
            pl.BlockSpec((_BM, 1), lambda i: (i, 0)),
            pl.BlockSpec((HALF, HID), lambda i: (0, 0)),
            pl.BlockSpec((HALF, HID), lambda i: (0, 0)),
            pl.BlockSpec((1, HID), lambda i: (0, 0)),
            pl.BlockSpec((HID, LW), lambda i: (0, 0)),
        ],
        out_specs=pl.BlockSpec((_BM, LW), lambda i: (i, 0)),
        out_shape=jax.ShapeDtypeStruct((N, LW), jnp.float32),
    )(agg_lo, agg_hi, dinv2, w1l, w1h, b1r, w2p)


def _final_body(z, s0, s1, dinv2, b2r, out):
    res = (z[...] + s0[...] + s1[...]) * dinv2[...]
    out[...] = res[:, :OUT_CH] + b2r[...]


def _final_call(z, s0, s1, dinv2, b2r):
    return pl.pallas_call(
        _final_body,
        grid=(N // _BM,),
        in_specs=[
            pl.BlockSpec((_BM, LW), lambda i: (i, 0)),
            pl.BlockSpec((_BM, LW), lambda i: (i, 0)),
            pl.BlockSpec((_BM, LW), lambda i: (i, 0)),
            pl.BlockSpec((_BM, 1), lambda i: (i, 0)),
            pl.BlockSpec((1, OUT_CH), lambda i: (0, 0)),
        ],
        out_specs=pl.BlockSpec((_BM, OUT_CH), lambda i: (i, 0)),
        out_shape=jax.ShapeDtypeStruct((N, OUT_CH), jnp.float32),
    )(z, s0, s1, dinv2, b2r)


def kernel(x, edge_index, W1, b1, W2, b2):
    ei = edge_index.astype(jnp.int32)
    pad_n = E_PAD - E
    pad_iota = jnp.arange(pad_n, dtype=jnp.int32)
    src = jnp.concatenate([ei[0], pad_iota % N])
    dst = jnp.concatenate([ei[1], N + pad_iota % PAD_ROWS])

    onehot = jnp.zeros((CHUNK, LW), jnp.float32).at[:, 0].set(1.0)
    zeros_np = jnp.zeros((NP_ROWS, LW), jnp.float32)
    w2p = jnp.zeros((HID, LW), jnp.float32).at[:, :OUT_CH].set(W2)

    dst_w = dst.reshape(NC * NS, _NW_CHUNKS, CHUNK)

    deg0, deg1 = _deg_kernel(dst_w, onehot, zeros_np)
    y_lo, y_hi, dinv2 = _prep_call(deg0, deg1, x)
    agg_lo, agg_hi = _agg1_kernel(y_lo, y_hi, src, dst)
    z = _main_call(agg_lo, agg_hi, dinv2,
                   W1[:HALF], W1[HALF:], b1.reshape(1, HID), w2p)
    s0, s1 = _agg2_kernel(z, zeros_np, src, dst)
    return _final_call(z, s0, s1, dinv2, b2.reshape(1, OUT_CH))


# speedup vs baseline: 2.4704x; 1.0023x over previous
import jax
import jax.numpy as jnp
from jax import lax
from jax.experimental import pallas as pl
from jax.experimental.pallas import tpu as pltpu
from jax.experimental.pallas import tpu_sc as plsc

N = 10000
E = 160000
IN_CH = 256
HID = 512
OUT_CH = 16
HALF = IN_CH // 2
LW = 128

NC, NS = 2, 16
CHUNK = 128
PAD_ROWS = 240
NP_ROWS = 10240
E_PAD = 163840

INIT_ROWS_PER_TILE = NP_ROWS // NS
ROWS_A, ROWS_B = 632, N - 15 * 632

_MESH = plsc.VectorSubcoreMesh(
    core_axis_name="c", subcore_axis_name="s", num_cores=NC, num_subcores=NS
)


def _al8(v):
    return pl.multiple_of(v, 8)


def _tile_rows_copy(s, get_src, get_dst):
    @pl.when(s < NS - 1)
    def _():
        pltpu.sync_copy(get_src(pl.ds(_al8(s * ROWS_A), ROWS_A)),
                        get_dst(pl.ds(_al8(s * ROWS_A), ROWS_A)))

    @pl.when(s == NS - 1)
    def _():
        pltpu.sync_copy(get_src(pl.ds((NS - 1) * ROWS_A, ROWS_B)),
                        get_dst(pl.ds((NS - 1) * ROWS_A, ROWS_B)))


def _init_zero(s, zeros_hbm, acc_sh):
    r0 = _al8(s * INIT_ROWS_PER_TILE)
    pltpu.sync_copy(zeros_hbm.at[pl.ds(r0, INIT_ROWS_PER_TILE)],
                    acc_sh.at[pl.ds(r0, INIT_ROWS_PER_TILE)])


_NW_CHUNKS = E_PAD // (NC * NS) // CHUNK
_NS_CHUNKS = E_PAD // NS // CHUNK


def _deg_body(dst_hbm, ones_hbm, zeros_hbm, deg0_hbm, deg1_hbm,
              idxs, ones_v, acc_sh, sem0, sem1):
    c = lax.axis_index("c")
    s = lax.axis_index("s")
    _init_zero(s, zeros_hbm, acc_sh)
    pltpu.sync_copy(ones_hbm, ones_v)
    pltpu.sync_copy(dst_hbm.at[c * NS + s], idxs)
    plsc.subcore_barrier()

    sems = (sem0, sem1)

    def wait(b):
        pltpu.make_async_copy(ones_v, acc_sh.at[idxs.at[0]], sems[b]).wait()

    def body(t, carry):
        for p in (0, 1):
            @pl.when(t > 0)
            def _():
                wait(p)

            pltpu.async_copy(ones_v, acc_sh.at[idxs.at[2 * t + p]],
                             sems[p], add=True)
        return carry

    lax.fori_loop(0, _NW_CHUNKS // 2, body, 0)
    wait(0)
    wait(1)
    plsc.subcore_barrier()

    @pl.when(c == 0)
    def _():
        _tile_rows_copy(s, lambda d: acc_sh.at[d], lambda d: deg0_hbm.at[d])

    @pl.when(c == 1)
    def _():
        _tile_rows_copy(s, lambda d: acc_sh.at[d], lambda d: deg1_hbm.at[d])


_deg_kernel = pl.kernel(
    _deg_body,
    out_type=(
        jax.ShapeDtypeStruct((N, LW), jnp.float32),
        jax.ShapeDtypeStruct((N, LW), jnp.float32),
    ),
    mesh=_MESH,
    scratch_types=[
        pltpu.VMEM((_NW_CHUNKS, CHUNK), jnp.int32),
        pltpu.VMEM((CHUNK, LW), jnp.float32),
        pltpu.VMEM_SHARED((NP_ROWS, LW), jnp.float32),
        pltpu.SemaphoreType.DMA,
        pltpu.SemaphoreType.DMA,
    ],
)


_CH = 64
_NR = 5


def _edge_sweep(n, base, srcf_hbm, dstf_hbm, gbufs, sbufs, rows,
                gi, si, sg, ss, acc_sh, gsrc_start, gsrc_wait):
    R = _NR

    def gidx_load(j, b):
        off = _al8(base + j * _CH)
        pltpu.async_copy(srcf_hbm.at[pl.ds(off, _CH)], gbufs[b], gi[b])

    def gidx_wait(b):
        pltpu.make_async_copy(srcf_hbm.at[pl.ds(0, _CH)], gbufs[b],
                              gi[b]).wait()

    def sidx_load(j, b):
        off = _al8(base + j * _CH)
        pltpu.async_copy(dstf_hbm.at[pl.ds(off, _CH)], sbufs[b], si[b])

    def sidx_wait(b):
        pltpu.make_async_copy(dstf_hbm.at[pl.ds(0, _CH)], sbufs[b],
                              si[b]).wait()

    def gather_start(b):
        gsrc_start(gbufs[b], rows[b], sg[b])

    def gather_wait(b):
        gsrc_wait(gbufs[b], rows[b], sg[b])

    def scat_start(b):
        pltpu.async_copy(rows[b], acc_sh.at[sbufs[b]], ss[b], add=True)

    def scat_wait(b):
        pltpu.make_async_copy(rows[b], acc_sh.at[sbufs[b]], ss[b]).wait()

    for k in range(R):
        gidx_load(k, k)
    for k in range(R - 1):
        sidx_load(k, k)
    for k in range(R - 1):
        gidx_wait(k)
        gather_start(k)

    def step(j, b):
        gather_wait(b)
        sidx_wait(b)
        scat_start(b)
        nb = (b + R - 1) % R

        @pl.when(j >= 1)
        def _():
            scat_wait(nb)

        @pl.when(j + R - 1 < n)
        def _():
            gidx_wait(nb)
            gather_start(nb)
            sidx_load(j + R - 1, nb)

        @pl.when(j + R < n)
        def _():
            gidx_load(j + R, b)

    def body(t, carry):
        for b in range(R):
            step(R * t + b, b)
        return carry

    lax.fori_loop(0, n // R, body, 0)
    scat_wait((n - 1) % R)


def _agg1_body(y_lo_hbm, y_hi_hbm, srcf_hbm, dstf_hbm, agg_lo_hbm, agg_hi_hbm,
               gbufs, sbufs, rows, acc_sh, gi, si, sg, ss):
    c = lax.axis_index("c")
    s = lax.axis_index("s")

    @pl.when(c == 0)
    def _():
        _tile_rows_copy(s, lambda d: y_lo_hbm.at[d], lambda d: acc_sh.at[d])

    @pl.when(c == 1)
    def _():
        _tile_rows_copy(s, lambda d: y_hi_hbm.at[d], lambda d: acc_sh.at[d])

    plsc.subcore_barrier()

    def gsrc_start(idx_ref, row_ref, sem):
        @pl.when(c == 0)
        def _():
            pltpu.async_copy(y_lo_hbm.at[idx_ref], row_ref, sem)

        @pl.when(c == 1)
        def _():
            pltpu.async_copy(y_hi_hbm.at[idx_ref], row_ref, sem)

    def gsrc_wait(idx_ref, row_ref, sem):
        pltpu.make_async_copy(y_lo_hbm.at[idx_ref], row_ref, sem).wait()

    base = s * (E_PAD // NS)
    _edge_sweep(E_PAD // NS // _CH, base, srcf_hbm, dstf_hbm,
                gbufs, sbufs, rows, gi, si, sg, ss,
                acc_sh, gsrc_start, gsrc_wait)
    plsc.subcore_barrier()

    @pl.when(c == 0)
    def _():
        _tile_rows_copy(s, lambda d: acc_sh.at[d], lambda d: agg_lo_hbm.at[d])

    @pl.when(c == 1)
    def _():
        _tile_rows_copy(s, lambda d: acc_sh.at[d], lambda d: agg_hi_hbm.at[d])


_agg1_kernel = pl.kernel(
    _agg1_body,
    out_type=(
        jax.ShapeDtypeStruct((N, HALF), jnp.float32),
        jax.ShapeDtypeStruct((N, HALF), jnp.float32),
    ),
    mesh=_MESH,
    scratch_types=[
        [pltpu.VMEM((_CH,), jnp.int32)] * _NR,
        [pltpu.VMEM((_CH,), jnp.int32)] * _NR,
        [pltpu.VMEM((_CH, HALF), jnp.float32)] * _NR,
        pltpu.VMEM_SHARED((NP_ROWS, HALF), jnp.float32),
        [pltpu.SemaphoreType.DMA] * _NR,
        [pltpu.SemaphoreType.DMA] * _NR,
        [pltpu.SemaphoreType.DMA] * _NR,
        [pltpu.SemaphoreType.DMA] * _NR,
    ],
)


def _agg2_body(z_hbm, zeros_hbm, srcf_hbm, dstf_hbm, s0_hbm, s1_hbm,
               gbufs, sbufs, rows, acc_sh, gi, si, sg, ss):
    c = lax.axis_index("c")
    s = lax.axis_index("s")
    _init_zero(s, zeros_hbm, acc_sh)
    plsc.subcore_barrier()

    def gsrc_start(idx_ref, row_ref, sem):
        pltpu.async_copy(z_hbm.at[idx_ref], row_ref, sem)

    def gsrc_wait(idx_ref, row_ref, sem):
        pltpu.make_async_copy(z_hbm.at[idx_ref], row_ref, sem).wait()

    base = (c * NS + s) * (E_PAD // (NC * NS))
    _edge_sweep(E_PAD // (NC * NS) // _CH, base, srcf_hbm, dstf_hbm,
                gbufs, sbufs, rows, gi, si, sg, ss,
                acc_sh, gsrc_start, gsrc_wait)
    plsc.subcore_barrier()

    @pl.when(c == 0)
    def _():
        _tile_rows_copy(s, lambda d: acc_sh.at[d], lambda d: s0_hbm.at[d])

    @pl.when(c == 1)
    def _():
        _tile_rows_copy(s, lambda d: acc_sh.at[d], lambda d: s1_hbm.at[d])


_agg2_kernel = pl.kernel(
    _agg2_body,
    out_type=(
        jax.ShapeDtypeStruct((N, LW), jnp.float32),
        jax.ShapeDtypeStruct((N, LW), jnp.float32),
    ),
    mesh=_MESH,
    scratch_types=[
        [pltpu.VMEM((_CH,), jnp.int32)] * _NR,
        [pltpu.VMEM((_CH,), jnp.int32)] * _NR,
        [pltpu.VMEM((_CH, LW), jnp.float32)] * _NR,
        pltpu.VMEM_SHARED((NP_ROWS, LW), jnp.float32),
        [pltpu.SemaphoreType.DMA] * _NR,
        [pltpu.SemaphoreType.DMA] * _NR,
        [pltpu.SemaphoreType.DMA] * _NR,
        [pltpu.SemaphoreType.DMA] * _NR,
    ],
)


_BM = 1000


def _prep_body(deg0, deg1, x, y_lo, y_hi, dinv2):
    deg = deg0[:, :1] + deg1[:, :1] + 1.0
    dinv = lax.rsqrt(deg)
    y = x[...] * dinv
    y_lo[...] = y[:, :HALF]
    y_hi[...] = y[:, HALF:]
    dinv2[...] = dinv


def _prep_call(deg0, deg1, x):
    return pl.pallas_call(
        _prep_body,
        grid=(N // _BM,),
        in_specs=[
            pl.BlockSpec((_BM, LW), lambda i: (i, 0)),
            pl.BlockSpec((_BM, LW), lambda i: (i, 0)),
            pl.BlockSpec((_BM, IN_CH), lambda i: (i, 0)),
        ],
        out_specs=[
            pl.BlockSpec((_BM, HALF), lambda i: (i, 0)),
            pl.BlockSpec((_BM, HALF), lambda i: (i, 0)),
            pl.BlockSpec((_BM, 1), lambda i: (i, 0)),
        ],
        out_shape=[
            jax.ShapeDtypeStruct((N, HALF), jnp.float32),
            jax.ShapeDtypeStruct((N, HALF), jnp.float32),
            jax.ShapeDtypeStruct((N, 1), jnp.float32),
        ],
    )(deg0, deg1, x)


def _bf(v):
    return v.astype(jnp.bfloat16)


def _main_body(agg_lo, agg_hi, dinv2, w1l, w1h, b1r, w2p, z_out):
    d = dinv2[...]
    h = (jnp.dot(_bf(agg_lo[...] * d), _bf(w1l[...]),
                 preferred_element_type=jnp.float32)
         + jnp.dot(_bf(agg_hi[...] * d), _bf(w1h[...]),
                   preferred_element_type=jnp.float32)
         + b1r[...])
    h = jnp.maximum(h, 0.0)
    z_out[...] = jnp.dot(_bf(h), _bf(w2p[...]),
                         preferred_element_type=jnp.float32) * d


def _main_call(agg_lo, agg_hi, dinv2, w1l, w1h, b1r, w2p):
    return pl.pallas_call(
        _main_body,
        grid=(N // _BM,),
        in_specs=[
            pl.BlockSpec((_BM, HALF), lambda i: (i, 0)),
            pl.BlockSpec((_BM, HALF), lambda i: (i, 0)),
            pl.BlockSpec((_BM, 1), lambda i: (i, 0)),
            pl.BlockSpec((HALF, HID), lambda i: (0, 0)),
            pl.BlockSpec((HALF, HID), lambda i: (0, 0)),
            pl.BlockSpec((1, HID), lambda i: (0, 0)),
            pl.BlockSpec((HID, LW), lambda i: (0, 0)),
        ],
        out_specs=pl.BlockSpec((_BM, LW), lambda i: (i, 0)),
        out_shape=jax.ShapeDtypeStruct((N, LW), jnp.float32),
    )(agg_lo, agg_hi, dinv2, w1l, w1h, b1r, w2p)


def _final_body(z, s0, s1, dinv2, b2r, out):
    res = (z[...] + s0[...] + s1[...]) * dinv2[...]
    out[...] = res[:, :OUT_CH] + b2r[...]


def _final_call(z, s0, s1, dinv2, b2r):
    return pl.pallas_call(
        _final_body,
        grid=(N // _BM,),
        in_specs=[
            pl.BlockSpec((_BM, LW), lambda i: (i, 0)),
            pl.BlockSpec((_BM, LW), lambda i: (i, 0)),
            pl.BlockSpec((_BM, LW), lambda i: (i, 0)),
            pl.BlockSpec((_BM, 1), lambda i: (i, 0)),
            pl.BlockSpec((1, OUT_CH), lambda i: (0, 0)),
        ],
        out_specs=pl.BlockSpec((_BM, OUT_CH), lambda i: (i, 0)),
        out_shape=jax.ShapeDtypeStruct((N, OUT_CH), jnp.float32),
    )(z, s0, s1, dinv2, b2r)


def kernel(x, edge_index, W1, b1, W2, b2):
    ei = edge_index.astype(jnp.int32)
    pad_n = E_PAD - E
    pad_iota = jnp.arange(pad_n, dtype=jnp.int32)
    src = jnp.concatenate([ei[0], pad_iota % N])
    dst = jnp.concatenate([ei[1], N + pad_iota % PAD_ROWS])

    onehot = jnp.zeros((CHUNK, LW), jnp.float32).at[:, 0].set(1.0)
    zeros_np = jnp.zeros((NP_ROWS, LW), jnp.float32)
    w2p = jnp.zeros((HID, LW), jnp.float32).at[:, :OUT_CH].set(W2)

    dst_w = dst.reshape(NC * NS, _NW_CHUNKS, CHUNK)

    deg0, deg1 = _deg_kernel(dst_w, onehot, zeros_np)
    y_lo, y_hi, dinv2 = _prep_call(deg0, deg1, x)
    agg_lo, agg_hi = _agg1_kernel(y_lo, y_hi, src, dst)
    z = _main_call(agg_lo, agg_hi, dinv2,
                   W1[:HALF], W1[HALF:], b1.reshape(1, HID), w2p)
    s0, s1 = _agg2_kernel(z, zeros_np, src, dst)
    return _final_call(z, s0, s1, dinv2, b2.reshape(1, OUT_CH))
